# Initial kernel scaffold; baseline (speedup 1.0000x reference)
import jax
import jax.numpy as jnp
from jax.experimental import pallas as pl


def kernel(x, edge_index, W1, b1, gamma, beta, W2, b2):
    raise NotImplementedError("write your pallas kernel here")


# speedup vs baseline: 14.7991x; 14.7991x over previous
import functools
import math

import jax
import jax.numpy as jnp
from jax import lax
from jax.experimental import pallas as pl
from jax.experimental.pallas import tpu as pltpu
from jax.experimental.pallas import tpu_sc as plsc

N = 10000
E = 320000
D_IN = 128
HID = 64
C = 40
C_PAD = 48
EPS = 1e-5
INV_S = 1.0 / math.sqrt(1.0 + EPS)

NC = 2
NS = 16
NW = NC * NS
EPW = E // NW
CH = 80
NCH = EPW // CH
N_PAD = 10240
NPT = N_PAD // NS


def _sc_mesh():
    return plsc.VectorSubcoreMesh(core_axis_name="c", subcore_axis_name="s")


@functools.partial(
    pl.kernel,
    out_type=jax.ShapeDtypeStruct((NC, N_PAD, 8), jnp.float32),
    mesh=_sc_mesh(),
    compiler_params=pltpu.CompilerParams(use_tc_tiling_on_sc=False),
    scratch_types=[
        pltpu.VMEM((CH,), jnp.int32),
        pltpu.VMEM((CH, 8), jnp.float32),
        pltpu.VMEM_SHARED((N_PAD, 8), jnp.float32),
    ],
)
def _deg_sc(dst_hbm, zeros_hbm, ones_hbm, out_hbm, idst, ones_v, acc):
    cid = lax.axis_index("c")
    sid = lax.axis_index("s")
    wid = sid * NC + cid
    pltpu.sync_copy(ones_hbm, ones_v)
    pltpu.sync_copy(zeros_hbm, acc.at[pl.ds(sid * NPT, NPT)])
    plsc.subcore_barrier()
    ebase = wid * EPW

    def body(c, carry):
        pltpu.sync_copy(dst_hbm.at[pl.ds(ebase + c * CH, CH)], idst)
        pltpu.sync_copy(ones_v, acc.at[idst], add=True)
        return carry

    lax.fori_loop(0, NCH, body, 0)
    plsc.subcore_barrier()
    pltpu.sync_copy(acc.at[pl.ds(sid * NPT, NPT)],
                    out_hbm.at[cid, pl.ds(sid * NPT, NPT)])


def _make_seg(W):
    @functools.partial(
        pl.kernel,
        out_type=jax.ShapeDtypeStruct((NC, N_PAD, W), jnp.float32),
        mesh=_sc_mesh(),
        compiler_params=pltpu.CompilerParams(use_tc_tiling_on_sc=False),
        scratch_types=[
            pltpu.VMEM((CH,), jnp.int32),
            pltpu.VMEM((CH,), jnp.int32),
            pltpu.VMEM((CH, W), jnp.float32),
            pltpu.VMEM_SHARED((N_PAD, W), jnp.float32),
            pltpu.SemaphoreType.DMA,
        ],
    )
    def seg(z_hbm, src_hbm, dst_hbm, zeros_hbm, out_hbm,
            isrc, idst, rows, acc, sem):
        cid = lax.axis_index("c")
        sid = lax.axis_index("s")
        wid = sid * NC + cid
        pltpu.sync_copy(zeros_hbm, acc.at[pl.ds(sid * NPT, NPT)])
        plsc.subcore_barrier()
        ebase = wid * EPW

        def body(c, carry):
            eoff = ebase + c * CH
            pltpu.sync_copy(src_hbm.at[pl.ds(eoff, CH)], isrc)
            pltpu.sync_copy(dst_hbm.at[pl.ds(eoff, CH)], idst)
            pltpu.async_copy(z_hbm.at[isrc], rows, sem).wait()
            pltpu.sync_copy(rows, acc.at[idst], add=True)
            return carry

        lax.fori_loop(0, NCH, body, 0)
        plsc.subcore_barrier()
        pltpu.sync_copy(acc.at[pl.ds(sid * NPT, NPT)],
                        out_hbm.at[cid, pl.ds(sid * NPT, NPT)])

    return seg


_seg_hid = _make_seg(HID)
_seg_out = _make_seg(C_PAD)


def _tc1_body(x_ref, w1_ref, degp_ref, z1_ref, dinv_ref):
    deg = degp_ref[0] + degp_ref[1] + 1.0
    dinv = lax.rsqrt(deg)
    dinv_ref[...] = dinv
    xw = jnp.dot(x_ref[...], w1_ref[...], preferred_element_type=jnp.float32)
    z1_ref[...] = xw * dinv


_tc1 = pl.pallas_call(
    _tc1_body,
    out_shape=[
        jax.ShapeDtypeStruct((N, HID), jnp.float32),
        jax.ShapeDtypeStruct((N, 1), jnp.float32),
    ],
)


def _tc2_body(a1p_ref, z1_ref, dinv_ref, g_ref, be_ref, b1_ref, w2_ref,
              z2_ref):
    dinv = dinv_ref[...]
    h1 = (a1p_ref[0] + a1p_ref[1] + z1_ref[...]) * dinv + b1_ref[...]
    h = jnp.maximum(g_ref[...] * (h1 * INV_S) + be_ref[...], 0.0)
    hw = jnp.dot(h, w2_ref[...], preferred_element_type=jnp.float32)
    z2_ref[...] = hw * dinv


_tc2 = pl.pallas_call(
    _tc2_body,
    out_shape=jax.ShapeDtypeStruct((N, C_PAD), jnp.float32),
)


def _tc3_body(a2p_ref, z2_ref, dinv_ref, b2_ref, out_ref):
    out_ref[...] = ((a2p_ref[0] + a2p_ref[1] + z2_ref[...]) * dinv_ref[...]
                    + b2_ref[...])


_tc3 = pl.pallas_call(
    _tc3_body,
    out_shape=jax.ShapeDtypeStruct((N, C_PAD), jnp.float32),
)


def kernel(x, edge_index, W1, b1, gamma, beta, W2, b2):
    src = edge_index[0].astype(jnp.int32)
    dst = edge_index[1].astype(jnp.int32)

    zeros8 = jnp.zeros((NPT, 8), jnp.float32)
    ones8 = jnp.ones((CH, 8), jnp.float32)
    zeros_hid = jnp.zeros((NPT, HID), jnp.float32)
    zeros_out = jnp.zeros((NPT, C_PAD), jnp.float32)
    w2p = jnp.pad(W2, ((0, 0), (0, C_PAD - C)))
    b2p = jnp.pad(b2, (0, C_PAD - C)).reshape(1, C_PAD)

    degp = _deg_sc(dst, zeros8, ones8)[:, :N]
    z1, dinv = _tc1(x, W1, degp[:, :, 0:1])
    a1p = _seg_hid(z1, src, dst, zeros_hid)[:, :N]
    z2 = _tc2(a1p, z1, dinv, gamma.reshape(1, HID),
              beta.reshape(1, HID), b1.reshape(1, HID), w2p)
    a2p = _seg_out(z2, src, dst, zeros_out)[:, :N]
    logits = _tc3(a2p, z2, dinv, b2p)
    return logits[:, :C]


# speedup vs baseline: 31.5858x; 2.1343x over previous
import functools
import math

import jax
import jax.numpy as jnp
from jax import lax
from jax.experimental import pallas as pl
from jax.experimental.pallas import tpu as pltpu
from jax.experimental.pallas import tpu_sc as plsc

N = 10000
E = 320000
D_IN = 128
HID = 64
C = 40
C_PAD = 48
EPS = 1e-5
INV_S = 1.0 / math.sqrt(1.0 + EPS)

NC = 2
NS = 16
NW = NC * NS
EPW = E // NW
CH = 80
NCH = EPW // CH
PIPE = NCH - 1
N_PAD = 10240
NPT = N_PAD // NS


def _sc_mesh():
    return plsc.VectorSubcoreMesh(core_axis_name="c", subcore_axis_name="s")


@functools.partial(
    pl.kernel,
    out_type=jax.ShapeDtypeStruct((NC, N_PAD, 8), jnp.float32),
    mesh=_sc_mesh(),
    compiler_params=pltpu.CompilerParams(use_tc_tiling_on_sc=False),
    scratch_types=[
        pltpu.VMEM((NCH, CH), jnp.int32),
        pltpu.VMEM((CH, 8), jnp.float32),
        pltpu.VMEM_SHARED((N_PAD, 8), jnp.float32),
        pltpu.SemaphoreType.DMA,
    ],
)
def _deg_sc(dst_hbm, zeros_hbm, ones_hbm, out_hbm, idst, ones_v, acc, sem):
    cid = lax.axis_index("c")
    sid = lax.axis_index("s")
    wid = sid * NC + cid
    pltpu.sync_copy(ones_hbm, ones_v)
    pltpu.sync_copy(dst_hbm.at[wid], idst)
    pltpu.sync_copy(zeros_hbm, acc.at[pl.ds(sid * NPT, NPT)])
    plsc.subcore_barrier()

    def body(c, carry):
        pltpu.async_copy(ones_v, acc.at[idst.at[c]], sem, add=True)

        @pl.when(c >= 1)
        def _():
            pltpu.make_async_copy(ones_v, acc.at[idst.at[0]], sem).wait()

        return carry

    lax.fori_loop(0, NCH, body, 0)
    pltpu.make_async_copy(ones_v, acc.at[idst.at[0]], sem).wait()
    plsc.subcore_barrier()
    pltpu.sync_copy(acc.at[pl.ds(sid * NPT, NPT)],
                    out_hbm.at[cid, pl.ds(sid * NPT, NPT)])


def _make_seg(W):
    @functools.partial(
        pl.kernel,
        out_type=jax.ShapeDtypeStruct((NC, N_PAD, W), jnp.float32),
        mesh=_sc_mesh(),
        compiler_params=pltpu.CompilerParams(use_tc_tiling_on_sc=False),
        scratch_types=[
            pltpu.VMEM((NCH, CH), jnp.int32),
            pltpu.VMEM((NCH, CH), jnp.int32),
            pltpu.VMEM((2, CH, W), jnp.float32),
            pltpu.VMEM_SHARED((N_PAD, W), jnp.float32),
            pltpu.SemaphoreType.DMA,
            pltpu.SemaphoreType.DMA,
            pltpu.SemaphoreType.DMA,
            pltpu.SemaphoreType.DMA,
        ],
    )
    def seg(z_hbm, src_hbm, dst_hbm, zeros_hbm, out_hbm,
            isrc, idst, rows, acc, sga, sgb, ssa, ssb):
        cid = lax.axis_index("c")
        sid = lax.axis_index("s")
        wid = sid * NC + cid
        pltpu.sync_copy(src_hbm.at[wid], isrc)
        pltpu.sync_copy(dst_hbm.at[wid], idst)
        pltpu.sync_copy(zeros_hbm, acc.at[pl.ds(sid * NPT, NPT)])
        plsc.subcore_barrier()

        pltpu.async_copy(z_hbm.at[isrc.at[0]], rows.at[0], sga)
        pltpu.async_copy(z_hbm.at[isrc.at[1]], rows.at[1], sgb)

        def body(i, carry):
            ca = 2 * i
            cb = 2 * i + 1
            pltpu.make_async_copy(z_hbm.at[isrc.at[ca]], rows.at[0],
                                  sga).wait()
            pltpu.async_copy(rows.at[0], acc.at[idst.at[ca]], ssa, add=True)
            pltpu.make_async_copy(z_hbm.at[isrc.at[cb]], rows.at[1],
                                  sgb).wait()
            pltpu.async_copy(rows.at[1], acc.at[idst.at[cb]], ssb, add=True)

            @pl.when(i < PIPE // 2 - 1)
            def _():
                pltpu.make_async_copy(rows.at[0], acc.at[idst.at[ca]],
                                      ssa).wait()
                pltpu.async_copy(z_hbm.at[isrc.at[ca + 2]], rows.at[0], sga)
                pltpu.make_async_copy(rows.at[1], acc.at[idst.at[cb]],
                                      ssb).wait()
                pltpu.async_copy(z_hbm.at[isrc.at[cb + 2]], rows.at[1], sgb)

            @pl.when(i == PIPE // 2 - 1)
            def _():
                pltpu.make_async_copy(rows.at[0], acc.at[idst.at[ca]],
                                      ssa).wait()
                pltpu.make_async_copy(rows.at[1], acc.at[idst.at[cb]],
                                      ssb).wait()

            return carry

        lax.fori_loop(0, PIPE // 2, body, 0)
        pltpu.async_copy(z_hbm.at[isrc.at[NCH - 1]], rows.at[0], sga).wait()
        pltpu.sync_copy(rows.at[0], acc.at[idst.at[NCH - 1]], add=True)
        plsc.subcore_barrier()
        pltpu.sync_copy(acc.at[pl.ds(sid * NPT, NPT)],
                        out_hbm.at[cid, pl.ds(sid * NPT, NPT)])

    return seg


_seg_hid = _make_seg(HID)
_seg_out = _make_seg(C_PAD)


def _tc1_body(x_ref, w1_ref, degp_ref, z1_ref, dinv_ref):
    deg = degp_ref[0] + degp_ref[1] + 1.0
    dinv = lax.rsqrt(deg)
    dinv_ref[...] = dinv
    xw = jnp.dot(x_ref[...], w1_ref[...], preferred_element_type=jnp.float32)
    z1_ref[...] = xw * dinv


_tc1 = pl.pallas_call(
    _tc1_body,
    out_shape=[
        jax.ShapeDtypeStruct((N, HID), jnp.float32),
        jax.ShapeDtypeStruct((N, 1), jnp.float32),
    ],
)


def _tc2_body(a1p_ref, z1_ref, dinv_ref, g_ref, be_ref, b1_ref, w2_ref,
              z2_ref):
    dinv = dinv_ref[...]
    h1 = (a1p_ref[0] + a1p_ref[1] + z1_ref[...]) * dinv + b1_ref[...]
    h = jnp.maximum(g_ref[...] * (h1 * INV_S) + be_ref[...], 0.0)
    hw = jnp.dot(h, w2_ref[...], preferred_element_type=jnp.float32)
    z2_ref[...] = hw * dinv


_tc2 = pl.pallas_call(
    _tc2_body,
    out_shape=jax.ShapeDtypeStruct((N, C_PAD), jnp.float32),
)


def _tc3_body(a2p_ref, z2_ref, dinv_ref, b2_ref, out_ref):
    out_ref[...] = ((a2p_ref[0] + a2p_ref[1] + z2_ref[...]) * dinv_ref[...]
                    + b2_ref[...])


_tc3 = pl.pallas_call(
    _tc3_body,
    out_shape=jax.ShapeDtypeStruct((N, C_PAD), jnp.float32),
)


def kernel(x, edge_index, W1, b1, gamma, beta, W2, b2):
    src = edge_index[0].astype(jnp.int32).reshape(NW, NCH, CH)
    dst = edge_index[1].astype(jnp.int32).reshape(NW, NCH, CH)

    zeros8 = jnp.zeros((NPT, 8), jnp.float32)
    ones8 = jnp.ones((CH, 8), jnp.float32)
    zeros_hid = jnp.zeros((NPT, HID), jnp.float32)
    zeros_out = jnp.zeros((NPT, C_PAD), jnp.float32)
    w2p = jnp.pad(W2, ((0, 0), (0, C_PAD - C)))
    b2p = jnp.pad(b2, (0, C_PAD - C)).reshape(1, C_PAD)

    degp = _deg_sc(dst, zeros8, ones8)[:, :N]
    z1, dinv = _tc1(x, W1, degp[:, :, 0:1])
    a1p = _seg_hid(z1, src, dst, zeros_hid)[:, :N]
    z2 = _tc2(a1p, z1, dinv, gamma.reshape(1, HID),
              beta.reshape(1, HID), b1.reshape(1, HID), w2p)
    a2p = _seg_out(z2, src, dst, zeros_out)[:, :N]
    logits = _tc3(a2p, z2, dinv, b2p)
    return logits[:, :C]
